# Initial kernel scaffold; baseline (speedup 1.0000x reference)
import jax
import jax.numpy as jnp
from jax.experimental import pallas as pl


def kernel(features, edge_index, cic_scores):
    raise NotImplementedError("write your pallas kernel here")


# speedup vs baseline: 3.9637x; 3.9637x over previous
import functools

import jax
import jax.numpy as jnp
from jax import lax
from jax.experimental import pallas as pl
from jax.experimental.pallas import tpu as pltpu
from jax.experimental.pallas import tpu_sc as plsc

N_NODES = 10000
N_PAD = 10240
D_FEAT = 128
K_SEEDS = 1000
N_EDGES = 320000
NW = 32
EDGES_PER_W = N_EDGES // NW
NODES_PER_W = N_PAD // NW
SENT = N_NODES
ROWS = N_PAD // 128


def _seed_body(scores_ref, seed_ref):
    s = scores_ref[...]
    b = lax.bitcast_convert_type(s, jnp.int32)
    ks = jnp.where(b >= 0, b, b ^ jnp.int32(0x7FFFFFFF))
    SIGN = jnp.int32(-(2**31))

    def bit_step(i, uprefix):
        bit = 31 - i
        bitval = lax.shift_left(jnp.int32(1), bit)
        ucand = uprefix | bitval
        scand = ucand ^ SIGN
        c = jnp.sum((ks >= scand).astype(jnp.int32))
        return jnp.where(c >= K_SEEDS, ucand, uprefix)

    uprefix = lax.fori_loop(0, 32, bit_step, jnp.int32(0))
    t = uprefix ^ SIGN
    gt = ks > t
    eq = ks == t
    need = jnp.float32(K_SEEDS) - jnp.sum(gt.astype(jnp.float32))
    eqf = eq.astype(jnp.float32)
    l_i = lax.broadcasted_iota(jnp.int32, (128, 128), 0)
    j_i = lax.broadcasted_iota(jnp.int32, (128, 128), 1)
    within = jnp.dot(eqf, (l_i < j_i).astype(jnp.float32),
                     preferred_element_type=jnp.float32)
    row_tot = jnp.sum(eqf, axis=1, keepdims=True)
    r_i = lax.broadcasted_iota(jnp.int32, (ROWS, ROWS), 0)
    c_i = lax.broadcasted_iota(jnp.int32, (ROWS, ROWS), 1)
    row_excl = jnp.dot((c_i < r_i).astype(jnp.float32), row_tot,
                       preferred_element_type=jnp.float32)
    rank = row_excl + within
    seed = gt | (eq & (rank < need))
    seed_ref[...] = seed.astype(jnp.int32)


_seed_call = pl.pallas_call(
    _seed_body,
    out_shape=jax.ShapeDtypeStruct((ROWS, 128), jnp.int32),
)


_sc_mesh = plsc.VectorSubcoreMesh(core_axis_name="c", subcore_axis_name="s")
_sc_params = pltpu.CompilerParams(needs_layout_passes=False)


@functools.partial(
    pl.kernel,
    mesh=_sc_mesh,
    compiler_params=_sc_params,
    out_type=jax.ShapeDtypeStruct((NW * N_PAD,), jnp.int32),
    scratch_types=[
        pltpu.VMEM((N_PAD,), jnp.int32),
        pltpu.VMEM((EDGES_PER_W,), jnp.int32),
        pltpu.VMEM((EDGES_PER_W,), jnp.int32),
    ],
)
def _tables_call(src_hbm, dst_hbm, tables_hbm, table_v, src_v, dst_v):
    wid = lax.axis_index("c") * 16 + lax.axis_index("s")

    def init_step(i, carry):
        table_v[pl.ds(i * 16, 16)] = jnp.full((16,), SENT, jnp.int32)
        return carry

    lax.fori_loop(0, N_PAD // 16, init_step, 0)

    base = wid * EDGES_PER_W
    pltpu.sync_copy(src_hbm.at[pl.ds(base, EDGES_PER_W)], src_v)
    pltpu.sync_copy(dst_hbm.at[pl.ds(base, EDGES_PER_W)], dst_v)

    iota16 = lax.iota(jnp.int32, 16)
    prev_idx = jnp.maximum(iota16 - 1, 0)

    def edge_step(j, carry):
        s = src_v[pl.ds(j * 16, 16)]
        d = dst_v[pl.ds(j * 16, 16)]
        kc = lax.shift_left(s, 15) | d
        ksort, _ = plsc.sort_key_val(kc, kc)
        ss = lax.shift_right_logical(ksort, 15)
        dd = ksort & jnp.int32(32767)
        prev = ss.at[prev_idx].get(mode="promise_in_bounds")
        leader = (iota16 == 0) | (ss != prev)
        old = plsc.load_gather(table_v, [ss])
        plsc.store_scatter(table_v, [ss], jnp.minimum(old, dd), mask=leader)
        return carry

    lax.fori_loop(0, EDGES_PER_W // 16, edge_step, 0)
    pltpu.sync_copy(table_v, tables_hbm.at[pl.ds(wid * N_PAD, N_PAD)])


@functools.partial(
    pl.kernel,
    mesh=_sc_mesh,
    compiler_params=_sc_params,
    out_type=(
        jax.ShapeDtypeStruct((N_PAD,), jnp.int32),
        jax.ShapeDtypeStruct((N_PAD,), jnp.int32),
    ),
    scratch_types=[
        pltpu.VMEM((NW * NODES_PER_W,), jnp.int32),
        pltpu.VMEM((NODES_PER_W,), jnp.int32),
        pltpu.VMEM((NODES_PER_W,), jnp.int32),
        pltpu.VMEM((NODES_PER_W,), jnp.int32),
    ],
)
def _merge_call(tables_hbm, seed_hbm, nextof_hbm, targets_hbm,
                buf_v, seed_v, next_v, tgt_v):
    wid = lax.axis_index("c") * 16 + lax.axis_index("s")
    base = wid * NODES_PER_W
    for c in range(NW):
        pltpu.sync_copy(tables_hbm.at[pl.ds(c * N_PAD + base, NODES_PER_W)],
                        buf_v.at[pl.ds(c * NODES_PER_W, NODES_PER_W)])
    pltpu.sync_copy(seed_hbm.at[pl.ds(base, NODES_PER_W)], seed_v)
    for j in range(NODES_PER_W // 16):
        m = buf_v[pl.ds(j * 16, 16)]
        for c in range(1, NW):
            m = jnp.minimum(m, buf_v[pl.ds(c * NODES_PER_W + j * 16, 16)])
        sd = seed_v[pl.ds(j * 16, 16)]
        hs = m < SENT
        part1 = (sd != 0) & hs
        next_v[pl.ds(j * 16, 16)] = m
        tgt_v[pl.ds(j * 16, 16)] = jnp.where(part1, m, jnp.int32(-1))
    pltpu.sync_copy(next_v, nextof_hbm.at[pl.ds(base, NODES_PER_W)])
    pltpu.sync_copy(tgt_v, targets_hbm.at[pl.ds(base, NODES_PER_W)])


@functools.partial(
    pl.kernel,
    mesh=_sc_mesh,
    compiler_params=_sc_params,
    out_type=jax.ShapeDtypeStruct((N_PAD,), jnp.int32),
    scratch_types=[
        pltpu.VMEM((N_PAD,), jnp.int32),
        pltpu.VMEM((N_PAD,), jnp.int32),
        pltpu.VMEM((NODES_PER_W,), jnp.int32),
        pltpu.VMEM((NODES_PER_W,), jnp.int32),
        pltpu.VMEM((NODES_PER_W,), jnp.int32),
    ],
)
def _mask_call(seed_hbm, nextof_hbm, targets_hbm, mask_hbm,
               tgt_v, reach_v, seed_v, next_v, mask_v):
    wid = lax.axis_index("c") * 16 + lax.axis_index("s")
    base = wid * NODES_PER_W
    pltpu.sync_copy(targets_hbm, tgt_v)
    pltpu.sync_copy(seed_hbm.at[pl.ds(base, NODES_PER_W)], seed_v)
    pltpu.sync_copy(nextof_hbm.at[pl.ds(base, NODES_PER_W)], next_v)

    def zero_step(i, carry):
        reach_v[pl.ds(i * 16, 16)] = jnp.zeros((16,), jnp.int32)
        return carry

    lax.fori_loop(0, N_PAD // 16, zero_step, 0)
    ones16 = jnp.ones((16,), jnp.int32)

    def scat_step(i, carry):
        t = tgt_v[pl.ds(i * 16, 16)]
        valid = t >= 0
        tt = jnp.maximum(t, 0)
        plsc.store_scatter(reach_v, [tt], ones16, mask=valid)
        return carry

    lax.fori_loop(0, N_PAD // 16, scat_step, 0)

    for j in range(NODES_PER_W // 16):
        r = reach_v[pl.ds(base + j * 16, 16)]
        sd = seed_v[pl.ds(j * 16, 16)]
        hs = next_v[pl.ds(j * 16, 16)] < SENT
        mask_v[pl.ds(j * 16, 16)] = (hs & ((sd != 0) | (r != 0))).astype(
            jnp.int32)
    pltpu.sync_copy(mask_v, mask_hbm.at[pl.ds(base, NODES_PER_W)])


_FEAT_BLK = 1000


def _feat_body(m_ref, f_ref, o_ref, b_ref):
    m = m_ref[...] != 0
    o_ref[...] = jnp.where(m, jnp.float32(0.0), f_ref[...])
    b_ref[...] = m


_feat_call = pl.pallas_call(
    _feat_body,
    grid=(N_NODES // _FEAT_BLK,),
    in_specs=[
        pl.BlockSpec((_FEAT_BLK, 1), lambda i: (i, 0)),
        pl.BlockSpec((_FEAT_BLK, D_FEAT), lambda i: (i, 0)),
    ],
    out_specs=[
        pl.BlockSpec((_FEAT_BLK, D_FEAT), lambda i: (i, 0)),
        pl.BlockSpec((_FEAT_BLK, 1), lambda i: (i, 0)),
    ],
    out_shape=[
        jax.ShapeDtypeStruct((N_NODES, D_FEAT), jnp.float32),
        jax.ShapeDtypeStruct((N_NODES, 1), jnp.bool_),
    ],
)


def kernel(features, edge_index, cic_scores):
    src = edge_index[0]
    dst = edge_index[1]
    scores_p = jnp.concatenate(
        [cic_scores, jnp.full((N_PAD - N_NODES,), -jnp.inf, cic_scores.dtype)]
    ).reshape(ROWS, 128)
    seed = _seed_call(scores_p).reshape(-1)
    tables = _tables_call(src, dst)
    nextof, targets = _merge_call(tables, seed)
    maskp = _mask_call(seed, nextof, targets)
    mask_col = maskp[:N_NODES, None]
    new_features, flags = _feat_call(mask_col, features)
    return new_features, flags.reshape(-1)


# speedup vs baseline: 4.6987x; 1.1854x over previous
import functools

import jax
import jax.numpy as jnp
from jax import lax
from jax.experimental import pallas as pl
from jax.experimental.pallas import tpu as pltpu
from jax.experimental.pallas import tpu_sc as plsc

N_NODES = 10000
N_PAD = 10240
D_FEAT = 128
K_SEEDS = 1000
N_EDGES = 320000
NW = 32
EDGES_PER_W = N_EDGES // NW
NODES_PER_W = N_PAD // NW
SENT = N_NODES
ROWS = N_PAD // 128


def _seed_body(scores_ref, seed_ref):
    s = scores_ref[...]
    b = lax.bitcast_convert_type(s, jnp.int32)
    ks = jnp.where(b >= 0, b, b ^ jnp.int32(0x7FFFFFFF))
    SIGN = jnp.int32(-(2**31))

    def bit_step(i, uprefix):
        bit = 31 - i
        bitval = lax.shift_left(jnp.int32(1), bit)
        ucand = uprefix | bitval
        scand = ucand ^ SIGN
        c = jnp.sum((ks >= scand).astype(jnp.int32))
        return jnp.where(c >= K_SEEDS, ucand, uprefix)

    uprefix = lax.fori_loop(0, 32, bit_step, jnp.int32(0))
    t = uprefix ^ SIGN
    gt = ks > t
    eq = ks == t
    need = jnp.float32(K_SEEDS) - jnp.sum(gt.astype(jnp.float32))
    eqf = eq.astype(jnp.float32)
    l_i = lax.broadcasted_iota(jnp.int32, (128, 128), 0)
    j_i = lax.broadcasted_iota(jnp.int32, (128, 128), 1)
    within = jnp.dot(eqf, (l_i < j_i).astype(jnp.float32),
                     preferred_element_type=jnp.float32)
    row_tot = jnp.sum(eqf, axis=1, keepdims=True)
    r_i = lax.broadcasted_iota(jnp.int32, (ROWS, ROWS), 0)
    c_i = lax.broadcasted_iota(jnp.int32, (ROWS, ROWS), 1)
    row_excl = jnp.dot((c_i < r_i).astype(jnp.float32), row_tot,
                       preferred_element_type=jnp.float32)
    rank = row_excl + within
    seed = gt | (eq & (rank < need))
    seed_ref[...] = seed.astype(jnp.int32)


_seed_call = pl.pallas_call(
    _seed_body,
    out_shape=jax.ShapeDtypeStruct((ROWS, 128), jnp.int32),
)


_sc_mesh = plsc.VectorSubcoreMesh(core_axis_name="c", subcore_axis_name="s")
_sc_params = pltpu.CompilerParams(needs_layout_passes=False)


@functools.partial(
    pl.kernel,
    mesh=_sc_mesh,
    compiler_params=_sc_params,
    out_type=jax.ShapeDtypeStruct((2 * N_PAD,), jnp.int32),
    scratch_types=[
        pltpu.VMEM((N_PAD,), jnp.int32),
        pltpu.VMEM((EDGES_PER_W,), jnp.int32),
        pltpu.VMEM((EDGES_PER_W,), jnp.int32),
        pltpu.VMEM((16, 640), jnp.int32),
        pltpu.VMEM_SHARED((16, 16, 640), jnp.int32),
        pltpu.SemaphoreType.DMA,
    ],
)
def _tables_call(src_hbm, dst_hbm, tables_hbm, table_v, src_v, dst_v,
                 tmp_v, shared_v, sem):
    cid = lax.axis_index("c")
    sid = lax.axis_index("s")
    wid = cid * 16 + sid

    def init_step(i, carry):
        table_v[pl.ds(i * 16, 16)] = jnp.full((16,), SENT, jnp.int32)
        return carry

    lax.fori_loop(0, N_PAD // 16, init_step, 0)

    base = wid * EDGES_PER_W
    pltpu.sync_copy(src_hbm.at[pl.ds(base, EDGES_PER_W)], src_v)
    pltpu.sync_copy(dst_hbm.at[pl.ds(base, EDGES_PER_W)], dst_v)

    iota16 = lax.iota(jnp.int32, 16)
    prev_idx = jnp.maximum(iota16 - 1, 0)

    def edge_step(j, carry):
        s = src_v[pl.ds(j * 16, 16)]
        d = dst_v[pl.ds(j * 16, 16)]
        kc = lax.shift_left(s, 15) | d
        ksort, _ = plsc.sort_key_val(kc, kc)
        ss = lax.shift_right_logical(ksort, 15)
        dd = ksort & jnp.int32(32767)
        prev = ss.at[prev_idx].get(mode="promise_in_bounds")
        leader = (iota16 == 0) | (ss != prev)
        old = plsc.load_gather(table_v, [ss])
        plsc.store_scatter(table_v, [ss], jnp.minimum(old, dd), mask=leader)
        return carry

    lax.fori_loop(0, EDGES_PER_W // 16, edge_step, 0)

    cps = [pltpu.async_copy(table_v.at[pl.ds(o * 640, 640)],
                            shared_v.at[sid, o], sem) for o in range(16)]
    for cp in cps:
        cp.wait()
    plsc.subcore_barrier()
    cps = [pltpu.async_copy(shared_v.at[w, sid], tmp_v.at[w], sem)
           for w in range(16)]
    for cp in cps:
        cp.wait()
    for j in range(640 // 16):
        m = tmp_v[0, pl.ds(j * 16, 16)]
        for w in range(1, 16):
            m = jnp.minimum(m, tmp_v[w, pl.ds(j * 16, 16)])
        table_v[pl.ds(j * 16, 16)] = m
    pltpu.sync_copy(table_v.at[pl.ds(0, 640)],
                    tables_hbm.at[pl.ds(cid * N_PAD + sid * 640, 640)])


@functools.partial(
    pl.kernel,
    mesh=_sc_mesh,
    compiler_params=_sc_params,
    out_type=jax.ShapeDtypeStruct((N_PAD,), jnp.int32),
    scratch_types=[
        pltpu.VMEM((N_PAD,), jnp.int32),
        pltpu.VMEM((N_PAD,), jnp.int32),
        pltpu.VMEM((N_PAD,), jnp.int32),
        pltpu.VMEM((N_PAD,), jnp.int32),
        pltpu.VMEM((NODES_PER_W,), jnp.int32),
        pltpu.SemaphoreType.DMA,
    ],
)
def _mask_call(tables_hbm, seed_hbm, mask_hbm,
               ta_v, tb_v, seed_v, reach_v, mask_v, sem):
    wid = lax.axis_index("c") * 16 + lax.axis_index("s")
    base = wid * NODES_PER_W
    cps = [
        pltpu.async_copy(tables_hbm.at[pl.ds(0, N_PAD)], ta_v, sem),
        pltpu.async_copy(tables_hbm.at[pl.ds(N_PAD, N_PAD)], tb_v, sem),
        pltpu.async_copy(seed_hbm, seed_v, sem),
    ]
    for cp in cps:
        cp.wait()

    def zero_step(i, carry):
        reach_v[pl.ds(i * 16, 16)] = jnp.zeros((16,), jnp.int32)
        return carry

    lax.fori_loop(0, N_PAD // 16, zero_step, 0)
    ones16 = jnp.ones((16,), jnp.int32)

    def scat_step(i, carry):
        nf = jnp.minimum(ta_v[pl.ds(i * 16, 16)], tb_v[pl.ds(i * 16, 16)])
        p1 = (seed_v[pl.ds(i * 16, 16)] != 0) & (nf < SENT)
        ti = jnp.where(p1, nf, 0)
        plsc.store_scatter(reach_v, [ti], ones16, mask=p1)
        return carry

    lax.fori_loop(0, N_PAD // 16, scat_step, 0)

    for j in range(NODES_PER_W // 16):
        nf = jnp.minimum(ta_v[pl.ds(base + j * 16, 16)],
                         tb_v[pl.ds(base + j * 16, 16)])
        hs = nf < SENT
        sd = seed_v[pl.ds(base + j * 16, 16)] != 0
        r = reach_v[pl.ds(base + j * 16, 16)] != 0
        mask_v[pl.ds(j * 16, 16)] = (hs & (sd | r)).astype(jnp.int32)
    pltpu.sync_copy(mask_v, mask_hbm.at[pl.ds(base, NODES_PER_W)])


_FEAT_BLK = 1000


def _feat_body(m_ref, f_ref, o_ref, b_ref):
    m = m_ref[...] != 0
    o_ref[...] = jnp.where(m, jnp.float32(0.0), f_ref[...])
    b_ref[...] = m


_feat_call = pl.pallas_call(
    _feat_body,
    grid=(N_NODES // _FEAT_BLK,),
    in_specs=[
        pl.BlockSpec((_FEAT_BLK, 1), lambda i: (i, 0)),
        pl.BlockSpec((_FEAT_BLK, D_FEAT), lambda i: (i, 0)),
    ],
    out_specs=[
        pl.BlockSpec((_FEAT_BLK, D_FEAT), lambda i: (i, 0)),
        pl.BlockSpec((_FEAT_BLK, 1), lambda i: (i, 0)),
    ],
    out_shape=[
        jax.ShapeDtypeStruct((N_NODES, D_FEAT), jnp.float32),
        jax.ShapeDtypeStruct((N_NODES, 1), jnp.bool_),
    ],
)


def kernel(features, edge_index, cic_scores):
    src = edge_index[0]
    dst = edge_index[1]
    scores_p = jnp.concatenate(
        [cic_scores, jnp.full((N_PAD - N_NODES,), -jnp.inf, cic_scores.dtype)]
    ).reshape(ROWS, 128)
    seed = _seed_call(scores_p).reshape(-1)
    tables = _tables_call(src, dst)
    maskp = _mask_call(tables, seed)
    mask_col = maskp[:N_NODES, None]
    new_features, flags = _feat_call(mask_col, features)
    return new_features, flags.reshape(-1)


# speedup vs baseline: 6.7027x; 1.4265x over previous
import functools

import jax
import jax.numpy as jnp
from jax import lax
from jax.experimental import pallas as pl
from jax.experimental.pallas import tpu as pltpu
from jax.experimental.pallas import tpu_sc as plsc

N_NODES = 10000
N_PAD = 10240
D_FEAT = 128
K_SEEDS = 1000
N_EDGES = 320000
NW = 32
EDGES_PER_W = N_EDGES // NW
NODES_PER_W = N_PAD // NW
SENT = N_NODES
ROWS = N_PAD // 128


def _seed_body(scores_ref, seed_ref):
    s = scores_ref[...]
    b = lax.bitcast_convert_type(s, jnp.int32)
    ks = jnp.where(b >= 0, b, b ^ jnp.int32(0x7FFFFFFF))
    SIGN = jnp.int32(-(2**31))

    def bit_step(i, uprefix):
        bit = 31 - i
        bitval = lax.shift_left(jnp.int32(1), bit)
        ucand = uprefix | bitval
        scand = ucand ^ SIGN
        c = jnp.sum((ks >= scand).astype(jnp.int32))
        return jnp.where(c >= K_SEEDS, ucand, uprefix)

    uprefix = lax.fori_loop(0, 32, bit_step, jnp.int32(0))
    t = uprefix ^ SIGN
    gt = ks > t
    eq = ks == t
    need = jnp.float32(K_SEEDS) - jnp.sum(gt.astype(jnp.float32))
    eqf = eq.astype(jnp.float32)
    l_i = lax.broadcasted_iota(jnp.int32, (128, 128), 0)
    j_i = lax.broadcasted_iota(jnp.int32, (128, 128), 1)
    within = jnp.dot(eqf, (l_i < j_i).astype(jnp.float32),
                     preferred_element_type=jnp.float32)
    row_tot = jnp.sum(eqf, axis=1, keepdims=True)
    r_i = lax.broadcasted_iota(jnp.int32, (ROWS, ROWS), 0)
    c_i = lax.broadcasted_iota(jnp.int32, (ROWS, ROWS), 1)
    row_excl = jnp.dot((c_i < r_i).astype(jnp.float32), row_tot,
                       preferred_element_type=jnp.float32)
    rank = row_excl + within
    seed = gt | (eq & (rank < need))
    seed_ref[...] = seed.astype(jnp.int32)


_seed_call = pl.pallas_call(
    _seed_body,
    out_shape=jax.ShapeDtypeStruct((ROWS, 128), jnp.int32),
)


_sc_mesh = plsc.VectorSubcoreMesh(core_axis_name="c", subcore_axis_name="s")
_sc_params = pltpu.CompilerParams(needs_layout_passes=False)


@functools.partial(
    pl.kernel,
    mesh=_sc_mesh,
    compiler_params=_sc_params,
    out_type=jax.ShapeDtypeStruct((2 * N_PAD,), jnp.int32),
    scratch_types=[
        pltpu.VMEM((N_PAD,), jnp.int32),
        pltpu.VMEM((N_PAD,), jnp.int32),
        pltpu.VMEM((2 * EDGES_PER_W,), jnp.int32),
        pltpu.VMEM((16, 640), jnp.int32),
        pltpu.VMEM_SHARED((16, 16, 640), jnp.int32),
        pltpu.SemaphoreType.DMA,
    ],
)
def _tables_call(ei_hbm, tables_hbm, t0_v, t1_v, buf_v, tmp_v, shared_v, sem):
    cid = lax.axis_index("c")
    sid = lax.axis_index("s")
    wid = cid * 16 + sid

    base = wid * EDGES_PER_W
    cps = [
        pltpu.async_copy(ei_hbm.at[pl.ds(base, EDGES_PER_W)],
                         buf_v.at[pl.ds(0, EDGES_PER_W)], sem),
        pltpu.async_copy(ei_hbm.at[pl.ds(N_EDGES + base, EDGES_PER_W)],
                         buf_v.at[pl.ds(EDGES_PER_W, EDGES_PER_W)], sem),
    ]

    sent16 = jnp.full((16,), SENT, jnp.int32)

    def init_step(i, carry):
        for u in range(4):
            t0_v[pl.ds(i * 64 + u * 16, 16)] = sent16
            t1_v[pl.ds(i * 64 + u * 16, 16)] = sent16
        return carry

    lax.fori_loop(0, N_PAD // 64, init_step, 0)
    for cp in cps:
        cp.wait()

    iota16 = lax.iota(jnp.int32, 16)
    prev_idx = jnp.maximum(iota16 - 1, 0)

    def rmw(j, table):
        s = buf_v[pl.ds(j * 16, 16)]
        d = buf_v[pl.ds(EDGES_PER_W + j * 16, 16)]
        kc = lax.shift_left(s, 15) | d
        ksort, _ = plsc.sort_key_val(kc, kc)
        ss = lax.shift_right_logical(ksort, 15)
        dd = ksort & jnp.int32(32767)
        prev = ss.at[prev_idx].get(mode="promise_in_bounds")
        leader = (iota16 == 0) | (ss != prev)
        old = plsc.load_gather(table, [ss])
        plsc.store_scatter(table, [ss], jnp.minimum(old, dd), mask=leader)

    def edge_step(j, carry):
        rmw(j * 2, t0_v)
        rmw(j * 2 + 1, t1_v)
        return carry

    n_pairs = (EDGES_PER_W // 16) // 2
    lax.fori_loop(0, n_pairs, edge_step, 0)
    rmw(EDGES_PER_W // 16 - 1, t0_v)

    def fold_step(j, carry):
        for u in range(2):
            off = j * 32 + u * 16
            t0_v[pl.ds(off, 16)] = jnp.minimum(t0_v[pl.ds(off, 16)],
                                               t1_v[pl.ds(off, 16)])
        return carry

    lax.fori_loop(0, N_PAD // 32, fold_step, 0)

    cps = [pltpu.async_copy(t0_v.at[pl.ds(o * 640, 640)],
                            shared_v.at[sid, o], sem) for o in range(16)]
    for cp in cps:
        cp.wait()
    plsc.subcore_barrier()
    cps = [pltpu.async_copy(shared_v.at[w, sid], tmp_v.at[w], sem)
           for w in range(16)]
    for cp in cps:
        cp.wait()

    def merge_step(j, carry):
        m = tmp_v[0, pl.ds(j * 16, 16)]
        for w in range(1, 16):
            m = jnp.minimum(m, tmp_v[w, pl.ds(j * 16, 16)])
        t0_v[pl.ds(j * 16, 16)] = m
        return carry

    lax.fori_loop(0, 640 // 16, merge_step, 0)
    pltpu.sync_copy(t0_v.at[pl.ds(0, 640)],
                    tables_hbm.at[pl.ds(cid * N_PAD + sid * 640, 640)])


@functools.partial(
    pl.kernel,
    mesh=_sc_mesh,
    compiler_params=_sc_params,
    out_type=jax.ShapeDtypeStruct((N_PAD,), jnp.int32),
    scratch_types=[
        pltpu.VMEM((N_PAD,), jnp.int32),
        pltpu.VMEM((N_PAD,), jnp.int32),
        pltpu.VMEM((N_PAD,), jnp.int32),
        pltpu.VMEM((N_PAD,), jnp.int32),
        pltpu.VMEM((NODES_PER_W,), jnp.int32),
        pltpu.SemaphoreType.DMA,
    ],
)
def _mask_call(tables_hbm, seed_hbm, mask_hbm,
               ta_v, tb_v, seed_v, reach_v, mask_v, sem):
    wid = lax.axis_index("c") * 16 + lax.axis_index("s")
    base = wid * NODES_PER_W
    cps = [
        pltpu.async_copy(tables_hbm.at[pl.ds(0, N_PAD)], ta_v, sem),
        pltpu.async_copy(tables_hbm.at[pl.ds(N_PAD, N_PAD)], tb_v, sem),
        pltpu.async_copy(seed_hbm, seed_v, sem),
    ]
    zero16 = jnp.zeros((16,), jnp.int32)

    def zero_step(i, carry):
        for u in range(4):
            reach_v[pl.ds(i * 64 + u * 16, 16)] = zero16
        return carry

    lax.fori_loop(0, N_PAD // 64, zero_step, 0)
    for cp in cps:
        cp.wait()
    ones16 = jnp.ones((16,), jnp.int32)

    def scat_step(i, carry):
        for u in range(4):
            off = (i * 4 + u) * 16
            nf = jnp.minimum(ta_v[pl.ds(off, 16)], tb_v[pl.ds(off, 16)])
            p1 = (seed_v[pl.ds(off, 16)] != 0) & (nf < SENT)
            ti = jnp.where(p1, nf, 0)
            plsc.store_scatter(reach_v, [ti], ones16, mask=p1)
        return carry

    lax.fori_loop(0, N_PAD // 64, scat_step, 0)

    def out_step(j, carry):
        nf = jnp.minimum(ta_v[pl.ds(base + j * 16, 16)],
                         tb_v[pl.ds(base + j * 16, 16)])
        hs = nf < SENT
        sd = seed_v[pl.ds(base + j * 16, 16)] != 0
        r = reach_v[pl.ds(base + j * 16, 16)] != 0
        mask_v[pl.ds(j * 16, 16)] = (hs & (sd | r)).astype(jnp.int32)
        return carry

    lax.fori_loop(0, NODES_PER_W // 16, out_step, 0)
    pltpu.sync_copy(mask_v, mask_hbm.at[pl.ds(base, NODES_PER_W)])


_FEAT_BLK = 1000


def _feat_body(m_ref, e0_ref, f_ref, o_ref):
    b = pl.program_id(0)
    m = (m_ref[...] != 0).astype(jnp.float32)
    qb = (b * _FEAT_BLK) // 128
    off = (b * _FEAT_BLK) % 128
    m9 = pltpu.roll(m, (ROWS - qb) % ROWS, 0)[0:9, :]
    r9 = lax.broadcasted_iota(jnp.int32, (_FEAT_BLK, 9), 0)
    q9 = lax.broadcasted_iota(jnp.int32, (_FEAT_BLK, 9), 1)
    g = (lax.shift_right_logical(off + r9, 7) == q9).astype(jnp.float32)
    t1 = jnp.dot(g, m9, preferred_element_type=jnp.float32)
    eb = pltpu.roll(e0_ref[...], off, 1)
    sel = jnp.sum(t1 * eb, axis=1, keepdims=True)
    o_ref[...] = jnp.where(sel > 0.0, jnp.float32(0.0), f_ref[...])


_feat_call = pl.pallas_call(
    _feat_body,
    grid=(N_NODES // _FEAT_BLK,),
    in_specs=[
        pl.BlockSpec((ROWS, 128), lambda i: (0, 0)),
        pl.BlockSpec((_FEAT_BLK, 128), lambda i: (0, 0)),
        pl.BlockSpec((_FEAT_BLK, D_FEAT), lambda i: (i, 0)),
    ],
    out_specs=pl.BlockSpec((_FEAT_BLK, D_FEAT), lambda i: (i, 0)),
    out_shape=jax.ShapeDtypeStruct((N_NODES, D_FEAT), jnp.float32),
)


def kernel(features, edge_index, cic_scores):
    ei_flat = edge_index.reshape(-1)
    scores_p = jnp.concatenate(
        [cic_scores, jnp.full((N_PAD - N_NODES,), -jnp.inf, cic_scores.dtype)]
    ).reshape(ROWS, 128)
    seed = _seed_call(scores_p).reshape(-1)
    tables = _tables_call(ei_flat)
    maskp = _mask_call(tables, seed)
    r1 = lax.broadcasted_iota(jnp.int32, (_FEAT_BLK, 128), 0)
    l1 = lax.broadcasted_iota(jnp.int32, (_FEAT_BLK, 128), 1)
    e0 = (l1 == (r1 & 127)).astype(jnp.float32)
    new_features = _feat_call(maskp.reshape(ROWS, 128), e0, features)
    return new_features, maskp[:N_NODES] != 0


# speedup vs baseline: 7.0495x; 1.0517x over previous
import functools

import jax
import jax.numpy as jnp
from jax import lax
from jax.experimental import pallas as pl
from jax.experimental.pallas import tpu as pltpu
from jax.experimental.pallas import tpu_sc as plsc

N_NODES = 10000
N_PAD = 10240
D_FEAT = 128
K_SEEDS = 1000
N_EDGES = 320000
NW = 32
EDGES_PER_W = N_EDGES // NW
NODES_PER_W = N_PAD // NW
SENT = N_NODES
ROWS = N_PAD // 128


def _seed_body(scores_ref, seed_ref):
    s = scores_ref[...]
    b = lax.bitcast_convert_type(s, jnp.int32)
    ks = jnp.where(b >= 0, b, b ^ jnp.int32(0x7FFFFFFF))
    SIGN = jnp.int32(-(2**31))

    def bit_step(i, uprefix):
        bit = 31 - i
        bitval = lax.shift_left(jnp.int32(1), bit)
        ucand = uprefix | bitval
        scand = ucand ^ SIGN
        c = jnp.sum((ks >= scand).astype(jnp.int32))
        return jnp.where(c >= K_SEEDS, ucand, uprefix)

    uprefix = lax.fori_loop(0, 32, bit_step, jnp.int32(0))
    t = uprefix ^ SIGN
    gt = ks > t
    eq = ks == t
    need = jnp.float32(K_SEEDS) - jnp.sum(gt.astype(jnp.float32))
    eqf = eq.astype(jnp.float32)
    l_i = lax.broadcasted_iota(jnp.int32, (128, 128), 0)
    j_i = lax.broadcasted_iota(jnp.int32, (128, 128), 1)
    within = jnp.dot(eqf, (l_i < j_i).astype(jnp.float32),
                     preferred_element_type=jnp.float32)
    row_tot = jnp.sum(eqf, axis=1, keepdims=True)
    r_i = lax.broadcasted_iota(jnp.int32, (ROWS, ROWS), 0)
    c_i = lax.broadcasted_iota(jnp.int32, (ROWS, ROWS), 1)
    row_excl = jnp.dot((c_i < r_i).astype(jnp.float32), row_tot,
                       preferred_element_type=jnp.float32)
    rank = row_excl + within
    seed = gt | (eq & (rank < need))
    seed_ref[...] = seed.astype(jnp.int32)


_seed_call = pl.pallas_call(
    _seed_body,
    out_shape=jax.ShapeDtypeStruct((ROWS, 128), jnp.int32),
)


_sc_mesh = plsc.VectorSubcoreMesh(core_axis_name="c", subcore_axis_name="s")
_sc_params = pltpu.CompilerParams(needs_layout_passes=False)


@functools.partial(
    pl.kernel,
    mesh=_sc_mesh,
    compiler_params=_sc_params,
    out_type=jax.ShapeDtypeStruct((2 * N_PAD,), jnp.int32),
    scratch_types=[
        pltpu.VMEM((N_PAD,), jnp.int32),
        pltpu.VMEM((N_PAD,), jnp.int32),
        pltpu.VMEM((2 * EDGES_PER_W,), jnp.int32),
        pltpu.VMEM((16, 640), jnp.int32),
        pltpu.VMEM_SHARED((16, 16, 640), jnp.int32),
        pltpu.SemaphoreType.DMA,
    ],
)
def _tables_call(ei_hbm, tables_hbm, t0_v, t1_v, buf_v, tmp_v, shared_v, sem):
    cid = lax.axis_index("c")
    sid = lax.axis_index("s")
    wid = cid * 16 + sid

    base = wid * EDGES_PER_W
    cps = [
        pltpu.async_copy(ei_hbm.at[pl.ds(base, EDGES_PER_W)],
                         buf_v.at[pl.ds(0, EDGES_PER_W)], sem),
        pltpu.async_copy(ei_hbm.at[pl.ds(N_EDGES + base, EDGES_PER_W)],
                         buf_v.at[pl.ds(EDGES_PER_W, EDGES_PER_W)], sem),
    ]

    sent16 = jnp.full((16,), SENT, jnp.int32)

    def init_step(i, carry):
        for u in range(4):
            t0_v[pl.ds(i * 64 + u * 16, 16)] = sent16
            t1_v[pl.ds(i * 64 + u * 16, 16)] = sent16
        return carry

    lax.fori_loop(0, N_PAD // 64, init_step, 0)
    for cp in cps:
        cp.wait()

    iota16 = lax.iota(jnp.int32, 16)
    prev_idx = jnp.maximum(iota16 - 1, 0)

    def rmw(j, table):
        s = buf_v[pl.ds(j * 16, 16)]
        d = buf_v[pl.ds(EDGES_PER_W + j * 16, 16)]
        kc = lax.shift_left(s, 15) | d
        ksort, _ = plsc.sort_key_val(kc, kc)
        ss = lax.shift_right_logical(ksort, 15)
        dd = ksort & jnp.int32(32767)
        prev = ss.at[prev_idx].get(mode="promise_in_bounds")
        leader = (iota16 == 0) | (ss != prev)
        old = plsc.load_gather(table, [ss])
        plsc.store_scatter(table, [ss], jnp.minimum(old, dd), mask=leader)

    def edge_step(j, carry):
        rmw(j * 2, t0_v)
        rmw(j * 2 + 1, t1_v)
        return carry

    n_pairs = (EDGES_PER_W // 16) // 2
    lax.fori_loop(0, n_pairs, edge_step, 0)
    rmw(EDGES_PER_W // 16 - 1, t0_v)

    def fold_step(j, carry):
        for u in range(2):
            off = j * 32 + u * 16
            t0_v[pl.ds(off, 16)] = jnp.minimum(t0_v[pl.ds(off, 16)],
                                               t1_v[pl.ds(off, 16)])
        return carry

    lax.fori_loop(0, N_PAD // 32, fold_step, 0)

    cps = [pltpu.async_copy(t0_v.at[pl.ds(o * 640, 640)],
                            shared_v.at[sid, o], sem) for o in range(16)]
    for cp in cps:
        cp.wait()
    plsc.subcore_barrier()
    cps = [pltpu.async_copy(shared_v.at[w, sid], tmp_v.at[w], sem)
           for w in range(16)]
    for cp in cps:
        cp.wait()

    def merge_step(j, carry):
        m = tmp_v[0, pl.ds(j * 16, 16)]
        for w in range(1, 16):
            m = jnp.minimum(m, tmp_v[w, pl.ds(j * 16, 16)])
        t0_v[pl.ds(j * 16, 16)] = m
        return carry

    lax.fori_loop(0, 640 // 16, merge_step, 0)
    pltpu.sync_copy(t0_v.at[pl.ds(0, 640)],
                    tables_hbm.at[pl.ds(cid * N_PAD + sid * 640, 640)])


@functools.partial(
    pl.kernel,
    mesh=_sc_mesh,
    compiler_params=_sc_params,
    out_type=jax.ShapeDtypeStruct((N_PAD,), jnp.int32),
    scratch_types=[
        pltpu.VMEM((640,), jnp.int32),
        pltpu.VMEM((640,), jnp.int32),
        pltpu.VMEM((640,), jnp.int32),
        pltpu.VMEM((640,), jnp.int32),
        pltpu.VMEM((16, 640), jnp.int32),
        pltpu.VMEM((NODES_PER_W,), jnp.int32),
        pltpu.VMEM((NODES_PER_W,), jnp.int32),
        pltpu.VMEM((NODES_PER_W,), jnp.int32),
        pltpu.VMEM((NODES_PER_W,), jnp.int32),
        pltpu.VMEM((NODES_PER_W,), jnp.int32),
        pltpu.VMEM_SHARED((16, 640), jnp.int32),
        pltpu.SemaphoreType.DMA,
    ],
)
def _mask_call(tables_hbm, seed_hbm, mask_hbm,
               ta_v, tb_v, seed_v, tgt_v, lists_v,
               oa_v, ob_v, os_v, reach_v, mask_v, shared_v, sem):
    cid = lax.axis_index("c")
    sid = lax.axis_index("s")
    wid = cid * 16 + sid
    sbase = sid * 640
    obase = wid * NODES_PER_W
    cps = [
        pltpu.async_copy(tables_hbm.at[pl.ds(sbase, 640)], ta_v, sem),
        pltpu.async_copy(tables_hbm.at[pl.ds(N_PAD + sbase, 640)], tb_v, sem),
        pltpu.async_copy(seed_hbm.at[pl.ds(sbase, 640)], seed_v, sem),
        pltpu.async_copy(tables_hbm.at[pl.ds(obase, NODES_PER_W)], oa_v, sem),
        pltpu.async_copy(tables_hbm.at[pl.ds(N_PAD + obase, NODES_PER_W)],
                         ob_v, sem),
        pltpu.async_copy(seed_hbm.at[pl.ds(obase, NODES_PER_W)], os_v, sem),
    ]
    zero16 = jnp.zeros((16,), jnp.int32)

    def rzero_step(i, carry):
        reach_v[pl.ds(i * 16, 16)] = zero16
        return carry

    lax.fori_loop(0, NODES_PER_W // 16, rzero_step, 0)
    for cp in cps:
        cp.wait()

    def scan_step(i, carry):
        for u in range(2):
            off = (i * 2 + u) * 16
            nf = jnp.minimum(ta_v[pl.ds(off, 16)], tb_v[pl.ds(off, 16)])
            p1 = (seed_v[pl.ds(off, 16)] != 0) & (nf < SENT)
            tgt_v[pl.ds(off, 16)] = jnp.where(p1, nf, jnp.int32(-1))
        return carry

    lax.fori_loop(0, 640 // 32, scan_step, 0)
    pltpu.sync_copy(tgt_v, shared_v.at[sid])
    plsc.subcore_barrier()
    cps = [pltpu.async_copy(shared_v.at[w], lists_v.at[w], sem)
           for w in range(16)]
    for cp in cps:
        cp.wait()
    ones16 = jnp.ones((16,), jnp.int32)
    lim = obase + NODES_PER_W

    for w in range(16):
        def reach_step(i, carry, w=w):
            t = lists_v[w, pl.ds(i * 16, 16)]
            inr = (t >= obase) & (t < lim)
            tt = jnp.where(inr, t - obase, 0)
            plsc.store_scatter(reach_v, [tt], ones16, mask=inr)
            return carry

        lax.fori_loop(0, 640 // 16, reach_step, 0)

    def out_step(j, carry):
        nf = jnp.minimum(oa_v[pl.ds(j * 16, 16)], ob_v[pl.ds(j * 16, 16)])
        hs = nf < SENT
        sd = os_v[pl.ds(j * 16, 16)] != 0
        r = reach_v[pl.ds(j * 16, 16)] != 0
        mask_v[pl.ds(j * 16, 16)] = (hs & (sd | r)).astype(jnp.int32)
        return carry

    lax.fori_loop(0, NODES_PER_W // 16, out_step, 0)
    pltpu.sync_copy(mask_v, mask_hbm.at[pl.ds(obase, NODES_PER_W)])


_FEAT_BLK = 1000


def _feat_body(m_ref, e0_ref, f_ref, o_ref):
    b = pl.program_id(0)
    m = (m_ref[...] != 0).astype(jnp.float32)
    qb = (b * _FEAT_BLK) // 128
    off = (b * _FEAT_BLK) % 128
    m9 = pltpu.roll(m, (ROWS - qb) % ROWS, 0)[0:9, :]
    r9 = lax.broadcasted_iota(jnp.int32, (_FEAT_BLK, 9), 0)
    q9 = lax.broadcasted_iota(jnp.int32, (_FEAT_BLK, 9), 1)
    g = (lax.shift_right_logical(off + r9, 7) == q9).astype(jnp.float32)
    t1 = jnp.dot(g, m9, preferred_element_type=jnp.float32)
    eb = pltpu.roll(e0_ref[...], off, 1)
    sel = jnp.sum(t1 * eb, axis=1, keepdims=True)
    o_ref[...] = jnp.where(sel > 0.0, jnp.float32(0.0), f_ref[...])


_feat_call = pl.pallas_call(
    _feat_body,
    grid=(N_NODES // _FEAT_BLK,),
    in_specs=[
        pl.BlockSpec((ROWS, 128), lambda i: (0, 0)),
        pl.BlockSpec((_FEAT_BLK, 128), lambda i: (0, 0)),
        pl.BlockSpec((_FEAT_BLK, D_FEAT), lambda i: (i, 0)),
    ],
    out_specs=pl.BlockSpec((_FEAT_BLK, D_FEAT), lambda i: (i, 0)),
    out_shape=jax.ShapeDtypeStruct((N_NODES, D_FEAT), jnp.float32),
)


def kernel(features, edge_index, cic_scores):
    ei_flat = edge_index.reshape(-1)
    scores_p = jnp.concatenate(
        [cic_scores, jnp.full((N_PAD - N_NODES,), -jnp.inf, cic_scores.dtype)]
    ).reshape(ROWS, 128)
    seed = _seed_call(scores_p).reshape(-1)
    tables = _tables_call(ei_flat)
    maskp = _mask_call(tables, seed)
    r1 = lax.broadcasted_iota(jnp.int32, (_FEAT_BLK, 128), 0)
    l1 = lax.broadcasted_iota(jnp.int32, (_FEAT_BLK, 128), 1)
    e0 = (l1 == (r1 & 127)).astype(jnp.float32)
    new_features = _feat_call(maskp.reshape(ROWS, 128), e0, features)
    return new_features, maskp[:N_NODES] != 0
